# Initial kernel scaffold; baseline (speedup 1.0000x reference)
import jax
import jax.numpy as jnp
from jax.experimental import pallas as pl


def kernel(ranks, suits, cards, rank_table, suit_table, card_table):
    raise NotImplementedError("write your pallas kernel here")


# speedup vs baseline: 42.2873x; 42.2873x over previous
import jax
import jax.numpy as jnp
from jax import lax
from jax.experimental import pallas as pl
from jax.experimental.pallas import tpu as pltpu
from jax.experimental.pallas import tpu_sc as plsc

BS = 16384
N_CARDS = 20
DIM = 128
NBINS = 128
NC, NS = 2, 16
NW = NC * NS
ROWS = BS // NW
GROUPS = ROWS // 16


def _sc_hist_body(ranks_hbm, suits_hbm, cards_hbm, counts_hbm,
                  ranks_v, suits_v, cards_v, counts_v):
    wid = lax.axis_index("s") * NC + lax.axis_index("c")

    idx_base = wid * (ROWS * N_CARDS)
    pltpu.sync_copy(ranks_hbm.at[pl.ds(idx_base, ROWS * N_CARDS)], ranks_v)
    pltpu.sync_copy(suits_hbm.at[pl.ds(idx_base, ROWS * N_CARDS)], suits_v)
    pltpu.sync_copy(cards_hbm.at[pl.ds(idx_base, ROWS * N_CARDS)], cards_v)

    zeros16 = jnp.zeros((16,), jnp.float32)

    def _zero(r, carry):
        for c in range(NBINS // 16):
            counts_v[pl.ds(r * NBINS + c * 16, 16)] = zeros16
        return carry

    lax.fori_loop(0, ROWS, _zero, 0)

    iota16 = lax.iota(jnp.int32, 16)
    ones16 = jnp.ones((16,), jnp.float32)

    def _group(g, carry):
        row16 = g * 16 + iota16
        gbase = row16 * N_CARDS
        abase = row16 * NBINS
        for ref_v, off in ((ranks_v, 0), (suits_v, 13), (cards_v, 17)):
            for j in range(N_CARDS):
                v = plsc.load_gather(ref_v, [gbase + j])
                m = v >= 0
                u = v + off if off else v
                plsc.addupdate_scatter(counts_v, [abase + u], ones16, mask=m)
        return carry

    lax.fori_loop(0, GROUPS, _group, 0)

    out_base = wid * (ROWS * NBINS)
    pltpu.sync_copy(counts_v, counts_hbm.at[pl.ds(out_base, ROWS * NBINS)])


_sc_hist = pl.kernel(
    _sc_hist_body,
    out_type=jax.ShapeDtypeStruct((BS * NBINS,), jnp.float32),
    mesh=plsc.VectorSubcoreMesh(core_axis_name="c", subcore_axis_name="s"),
    compiler_params=pltpu.CompilerParams(needs_layout_passes=False),
    scratch_types=[
        pltpu.VMEM((ROWS * N_CARDS,), jnp.int32),
        pltpu.VMEM((ROWS * N_CARDS,), jnp.int32),
        pltpu.VMEM((ROWS * N_CARDS,), jnp.int32),
        pltpu.VMEM((ROWS * NBINS,), jnp.float32),
    ],
)


def _matmul_body(c_ref, t_ref, o_ref):
    o_ref[...] = jnp.dot(c_ref[...], t_ref[...],
                         preferred_element_type=jnp.float32)


_BLK = 2048

_matmul = pl.pallas_call(
    _matmul_body,
    grid=(BS // _BLK,),
    in_specs=[
        pl.BlockSpec((_BLK, NBINS), lambda i: (i, 0)),
        pl.BlockSpec((NBINS, DIM), lambda i: (0, 0)),
    ],
    out_specs=pl.BlockSpec((_BLK, DIM), lambda i: (i, 0)),
    out_shape=jax.ShapeDtypeStruct((BS, DIM), jnp.float32),
)


def kernel(ranks, suits, cards, rank_table, suit_table, card_table):
    tpad = jnp.concatenate(
        [rank_table, suit_table, card_table,
         jnp.zeros((NBINS - 69, DIM), jnp.float32)], axis=0)
    counts = _sc_hist(ranks.reshape(-1), suits.reshape(-1), cards.reshape(-1))
    return _matmul(counts.reshape(BS, NBINS), tpad)


# speedup vs baseline: 78.6505x; 1.8599x over previous
import jax
import jax.numpy as jnp
from jax import lax
from jax.experimental import pallas as pl
from jax.experimental.pallas import tpu as pltpu
from jax.experimental.pallas import tpu_sc as plsc

BS = 16384
N_CARDS = 20
DIM = 128
NBINS = 72
NSLOT = 3 * N_CARDS
NC, NS = 2, 16
NW = NC * NS
ROWS = BS // NW
GROUPS = ROWS // 16
_OFFS = (0,) * N_CARDS + (13,) * N_CARDS + (17,) * N_CARDS


def _sc_hist_body(idx_hbm, counts_hbm, idx_v, counts_v):
    wid = lax.axis_index("s") * NC + lax.axis_index("c")
    base = wid * ROWS

    pltpu.sync_copy(idx_hbm.at[:, pl.ds(base, ROWS)], idx_v)

    zeros16 = jnp.zeros((16,), jnp.float32)

    def _zero(r, carry):
        for c in range(ROWS // (16 * 8)):
            for k in range(8):
                counts_v[r, pl.ds(c * 128 + k * 16, 16)] = zeros16
        return carry

    lax.fori_loop(0, NBINS, _zero, 0)

    iota16 = lax.iota(jnp.int32, 16)
    ones16 = jnp.ones((16,), jnp.float32)

    def _group(g, carry):
        row16 = g * 16 + iota16
        for j in range(NSLOT):
            v = idx_v[j, pl.ds(g * 16, 16)]
            m = v >= _OFFS[j]
            plsc.addupdate_scatter(counts_v, [v, row16], ones16, mask=m)
        return carry

    lax.fori_loop(0, GROUPS, _group, 0)

    pltpu.sync_copy(counts_v, counts_hbm.at[:, pl.ds(base, ROWS)])


_sc_hist = pl.kernel(
    _sc_hist_body,
    out_type=jax.ShapeDtypeStruct((NBINS, BS), jnp.float32),
    mesh=plsc.VectorSubcoreMesh(core_axis_name="c", subcore_axis_name="s"),
    compiler_params=pltpu.CompilerParams(needs_layout_passes=False),
    scratch_types=[
        pltpu.VMEM((NSLOT, ROWS), jnp.int32),
        pltpu.VMEM((NBINS, ROWS), jnp.float32),
    ],
)


def _matmul_body(c_ref, t_ref, o_ref):
    o_ref[...] = lax.dot_general(c_ref[...], t_ref[...],
                                 (((0,), (0,)), ((), ())),
                                 preferred_element_type=jnp.float32)


_BLK = 2048

_matmul = pl.pallas_call(
    _matmul_body,
    grid=(BS // _BLK,),
    in_specs=[
        pl.BlockSpec((NBINS, _BLK), lambda i: (0, i)),
        pl.BlockSpec((NBINS, DIM), lambda i: (0, 0)),
    ],
    out_specs=pl.BlockSpec((_BLK, DIM), lambda i: (i, 0)),
    out_shape=jax.ShapeDtypeStruct((BS, DIM), jnp.float32),
)


def kernel(ranks, suits, cards, rank_table, suit_table, card_table):
    idx = jnp.concatenate(
        [ranks.T, suits.T + 13, cards.T + 17], axis=0)
    tpad = jnp.concatenate(
        [rank_table, suit_table, card_table,
         jnp.zeros((NBINS - 69, DIM), jnp.float32)], axis=0)
    counts = _sc_hist(idx)
    return _matmul(counts, tpad)


# speedup vs baseline: 104.6357x; 1.3304x over previous
import jax
import jax.numpy as jnp
from jax import lax
from jax.experimental import pallas as pl
from jax.experimental.pallas import tpu as pltpu
from jax.experimental.pallas import tpu_sc as plsc

BS = 16384
N_CARDS = 20
DIM = 128
NBINS = 80
SUIT_OFF = 16
CARD_OFF = 24
NSLOT = 3 * N_CARDS
NC, NS = 2, 16
NW = NC * NS
ROWS = BS // NW
GROUPS = ROWS // 16
HALF = GROUPS // 2
_OFFS = (0,) * N_CARDS + (SUIT_OFF,) * N_CARDS + (CARD_OFF,) * N_CARDS
_CHUNK = 10


def _sc_hist_body(idx_hbm, counts_hbm, idx_v, counts_v, sem_in, sem_o1, sem_o2):
    wid = lax.axis_index("s") * NC + lax.axis_index("c")
    base = wid * ROWS

    stage = pltpu.make_async_copy(idx_hbm.at[:, pl.ds(base, ROWS)], idx_v,
                                  sem_in)
    stage.start()

    zeros16 = jnp.zeros((16,), jnp.float32)

    def _zero(r, carry):
        for c in range(ROWS // (16 * 8)):
            for k in range(8):
                counts_v[r, pl.ds(c * 128 + k * 16, 16)] = zeros16
        return carry

    lax.fori_loop(0, NBINS, _zero, 0)
    stage.wait()

    iota16 = lax.iota(jnp.int32, 16)
    ones16 = jnp.ones((16,), jnp.float32)

    def _group(g, carry):
        row16 = g * 16 + iota16
        for c0 in range(0, NSLOT, _CHUNK):
            vs = [idx_v[j, pl.ds(g * 16, 16)] for j in range(c0, c0 + _CHUNK)]
            ms = [vs[k] >= _OFFS[c0 + k] for k in range(_CHUNK)]
            for k in range(_CHUNK):
                plsc.addupdate_scatter(counts_v, [vs[k], row16], ones16,
                                       mask=ms[k])
        return carry

    lax.fori_loop(0, HALF, _group, 0)
    out1 = pltpu.make_async_copy(
        counts_v.at[:, pl.ds(0, ROWS // 2)],
        counts_hbm.at[:, pl.ds(base, ROWS // 2)], sem_o1)
    out1.start()

    lax.fori_loop(HALF, GROUPS, _group, 0)
    out2 = pltpu.make_async_copy(
        counts_v.at[:, pl.ds(ROWS // 2, ROWS // 2)],
        counts_hbm.at[:, pl.ds(base + ROWS // 2, ROWS // 2)], sem_o2)
    out2.start()

    out1.wait()
    out2.wait()


_sc_hist = pl.kernel(
    _sc_hist_body,
    out_type=jax.ShapeDtypeStruct((NBINS, BS), jnp.float32),
    mesh=plsc.VectorSubcoreMesh(core_axis_name="c", subcore_axis_name="s"),
    compiler_params=pltpu.CompilerParams(needs_layout_passes=False),
    scratch_types=[
        pltpu.VMEM((NSLOT, ROWS), jnp.int32),
        pltpu.VMEM((NBINS, ROWS), jnp.float32),
        pltpu.SemaphoreType.DMA,
        pltpu.SemaphoreType.DMA,
        pltpu.SemaphoreType.DMA,
    ],
)


def _matmul_body(c_ref, rank_ref, suit_ref, card_ref, o_ref, t_ref):
    @pl.when(pl.program_id(0) == 0)
    def _assemble():
        t_ref[...] = jnp.zeros((NBINS, DIM), jnp.float32)
        t_ref[0:13, :] = rank_ref[...]
        t_ref[SUIT_OFF:SUIT_OFF + 4, :] = suit_ref[...]
        t_ref[CARD_OFF:CARD_OFF + 52, :] = card_ref[...]

    o_ref[...] = lax.dot_general(c_ref[...], t_ref[...],
                                 (((0,), (0,)), ((), ())),
                                 preferred_element_type=jnp.float32)


_BLK = 2048

_matmul = pl.pallas_call(
    _matmul_body,
    grid=(BS // _BLK,),
    in_specs=[
        pl.BlockSpec((NBINS, _BLK), lambda i: (0, i)),
        pl.BlockSpec((13, DIM), lambda i: (0, 0)),
        pl.BlockSpec((4, DIM), lambda i: (0, 0)),
        pl.BlockSpec((52, DIM), lambda i: (0, 0)),
    ],
    out_specs=pl.BlockSpec((_BLK, DIM), lambda i: (i, 0)),
    out_shape=jax.ShapeDtypeStruct((BS, DIM), jnp.float32),
    scratch_shapes=[pltpu.VMEM((NBINS, DIM), jnp.float32)],
)


def kernel(ranks, suits, cards, rank_table, suit_table, card_table):
    idx = jnp.concatenate(
        [ranks.T, suits.T + SUIT_OFF, cards.T + CARD_OFF], axis=0)
    counts = _sc_hist(idx)
    return _matmul(counts, rank_table, suit_table, card_table)
